# baseline (speedup 1.0000x reference)
import math

import jax
import jax.numpy as jnp
from jax.experimental import pallas as pl
from jax.experimental.pallas import tpu as pltpu

_N = 16384
_SIZE = 1000
_SMOOTH = 0.1
_CONF = 1.0 - _SMOOTH
_S = _SMOOTH / (_SIZE - 2)
_K = (_SIZE - 2) * _S * math.log(_S) + _CONF * math.log(_CONF)

_SAMPLES_PER_BLOCK = 4096
_GRID = _N // _SAMPLES_PER_BLOCK


def _tc_body(y_ref, tgt_ref, out_ref):
    i = pl.program_id(0)

    @pl.when(i == 0)
    def _init():
        out_ref[...] = jnp.zeros((1, 1), jnp.float32)

    y = y_ref[...]
    tgt = tgt_ref[...]
    valid = (tgt != 0)
    colsum = jnp.sum(y, axis=0, keepdims=True)
    x0 = y[0:1, :]
    classes = jax.lax.broadcasted_iota(jnp.int32, y.shape, 0)
    pick = jnp.sum(jnp.where(classes == tgt, y, 0.0), axis=0, keepdims=True)
    per_col = _K - _S * colsum + _S * x0 - (_CONF - _S) * pick
    out_ref[...] += jnp.sum(jnp.where(valid, per_col, 0.0)).reshape(1, 1)


def kernel(x, target):
    y = x.T
    tgt = target.astype(jnp.int32).reshape(1, _N)
    out = pl.pallas_call(
        _tc_body,
        grid=(_GRID,),
        in_specs=[
            pl.BlockSpec((_SIZE, _SAMPLES_PER_BLOCK), lambda i: (0, i)),
            pl.BlockSpec((1, _SAMPLES_PER_BLOCK), lambda i: (0, i)),
        ],
        out_specs=pl.BlockSpec((1, 1), lambda i: (0, 0)),
        out_shape=jax.ShapeDtypeStruct((1, 1), jnp.float32),
        compiler_params=pltpu.CompilerParams(
            dimension_semantics=("arbitrary",),
        ),
    )(y, tgt)
    return out[0, 0]


# speedup vs baseline: 1.0415x; 1.0415x over previous
import math

import jax
import jax.numpy as jnp
from jax.experimental import pallas as pl
from jax.experimental.pallas import tpu as pltpu

_N = 16384
_SIZE = 1000
_SMOOTH = 0.1
_CONF = 1.0 - _SMOOTH
_S = _SMOOTH / (_SIZE - 2)
_K = (_SIZE - 2) * _S * math.log(_S) + _CONF * math.log(_CONF)

_SAMPLES_PER_BLOCK = 2048
_GRID = _N // _SAMPLES_PER_BLOCK


def _tc_body(y_ref, tgt_ref, out_ref):
    i = pl.program_id(0)

    @pl.when(i == 0)
    def _init():
        out_ref[...] = jnp.zeros((1, 1), jnp.float32)

    y = y_ref[...]
    tgt = tgt_ref[...]
    valid = (tgt != 0)
    colsum = jnp.sum(y, axis=0, keepdims=True)
    x0 = y[0:1, :]
    classes = jax.lax.broadcasted_iota(jnp.int32, y.shape, 0)
    pick = jnp.sum(jnp.where(classes == tgt, y, 0.0), axis=0, keepdims=True)
    per_col = _K - _S * colsum + _S * x0 - (_CONF - _S) * pick
    out_ref[...] += jnp.sum(jnp.where(valid, per_col, 0.0)).reshape(1, 1)


def kernel(x, target):
    y = x.T
    tgt = target.astype(jnp.int32).reshape(1, _N)
    out = pl.pallas_call(
        _tc_body,
        grid=(_GRID,),
        in_specs=[
            pl.BlockSpec((_SIZE, _SAMPLES_PER_BLOCK), lambda i: (0, i)),
            pl.BlockSpec((1, _SAMPLES_PER_BLOCK), lambda i: (0, i)),
        ],
        out_specs=pl.BlockSpec((1, 1), lambda i: (0, 0)),
        out_shape=jax.ShapeDtypeStruct((1, 1), jnp.float32),
        compiler_params=pltpu.CompilerParams(
            dimension_semantics=("arbitrary",),
        ),
    )(y, tgt)
    return out[0, 0]
